# Initial kernel scaffold; baseline (speedup 1.0000x reference)
import jax
import jax.numpy as jnp
from jax.experimental import pallas as pl


def kernel(preference_vector, codebooks):
    raise NotImplementedError("write your pallas kernel here")


# speedup vs baseline: 11.7227x; 11.7227x over previous
import jax
import jax.numpy as jnp
from jax.experimental import pallas as pl
from jax.experimental.pallas import tpu as pltpu

NUM_LEVELS_ = 8
K_ = 8192
D_ = 256
B_ = 4096
BT_ = 256
NBT_ = B_ // BT_

_HI = jax.lax.Precision.HIGHEST


def _rvq_body(r_in_ref, cb_ref, ids_ref, r_out_ref,
              r_scratch, cnorm_scratch):
    l = pl.program_id(0)
    b = pl.program_id(1)

    @pl.when(l == 0)
    def _init_residual():
        r_scratch[pl.ds(b * BT_, BT_), :] = r_in_ref[...]

    cb = cb_ref[0]

    @pl.when(b == 0)
    def _level_norms():
        ones = jnp.ones((1, D_), jnp.float32)
        cnorm_scratch[...] = jax.lax.dot_general(
            ones, cb * cb, (((1,), (1,)), ((), ())),
            precision=_HI, preferred_element_type=jnp.float32)

    r = r_scratch[pl.ds(b * BT_, BT_), :]
    rc = jax.lax.dot_general(
        r, cb, (((1,), (1,)), ((), ())),
        precision=_HI, preferred_element_type=jnp.float32)
    s = cnorm_scratch[...] - 2.0 * rc

    m = jnp.min(s, axis=1, keepdims=True)
    kiota = jax.lax.broadcasted_iota(jnp.int32, (BT_, K_), 1)
    idx = jnp.min(jnp.where(s == m, kiota, K_), axis=1)

    onehot = (kiota == idx[:, None]).astype(jnp.float32)
    chosen = jax.lax.dot_general(
        onehot, cb, (((1,), (0,)), ((), ())),
        precision=_HI, preferred_element_type=jnp.float32)
    r_new = r - chosen
    r_scratch[pl.ds(b * BT_, BT_), :] = r_new
    r_out_ref[...] = r_new
    ids_ref[...] = idx.reshape(1, 1, BT_)


def kernel(preference_vector, codebooks):
    ids_lb, residual = pl.pallas_call(
        _rvq_body,
        grid=(NUM_LEVELS_, NBT_),
        in_specs=[
            pl.BlockSpec((BT_, D_), lambda l, b: (b, 0)),
            pl.BlockSpec((1, K_, D_), lambda l, b: (l, 0, 0)),
        ],
        out_specs=[
            pl.BlockSpec((1, 1, BT_), lambda l, b: (l, 0, b)),
            pl.BlockSpec((BT_, D_), lambda l, b: (b, 0)),
        ],
        out_shape=[
            jax.ShapeDtypeStruct((NUM_LEVELS_, 1, B_), jnp.int32),
            jax.ShapeDtypeStruct((B_, D_), jnp.float32),
        ],
        scratch_shapes=[
            pltpu.VMEM((B_, D_), jnp.float32),
            pltpu.VMEM((1, K_), jnp.float32),
        ],
    )(preference_vector, codebooks)
    ids = ids_lb.reshape(NUM_LEVELS_, B_).T
    return ids, residual


# speedup vs baseline: 17.9983x; 1.5353x over previous
import jax
import jax.numpy as jnp
from jax.experimental import pallas as pl
from jax.experimental.pallas import tpu as pltpu

NUM_LEVELS_ = 8
K_ = 8192
D_ = 256
B_ = 4096
BT_ = 256
NBT_ = B_ // BT_

_HI = jax.lax.Precision.HIGHEST


def _nt(a, b):
    return jax.lax.dot_general(a, b, (((1,), (1,)), ((), ())),
                               preferred_element_type=jnp.float32)


def _nn(a, b):
    return jax.lax.dot_general(a, b, (((1,), (0,)), ((), ())),
                               preferred_element_type=jnp.float32)


def _split3(x):
    x0 = x.astype(jnp.bfloat16)
    rem = x - x0.astype(jnp.float32)
    x1 = rem.astype(jnp.bfloat16)
    x2 = (rem - x1.astype(jnp.float32)).astype(jnp.bfloat16)
    return x0, x1, x2


def _rvq_body(r_in_ref, cb_ref, ids_ref, r_out_ref,
              r_scratch, cnorm_scratch, c0_s, c1_s, c2_s):
    l = pl.program_id(0)
    b = pl.program_id(1)

    @pl.when(l == 0)
    def _init_residual():
        r_scratch[pl.ds(b * BT_, BT_), :] = r_in_ref[...]

    @pl.when(b == 0)
    def _per_level_prep():
        cb = cb_ref[0]
        ones = jnp.ones((1, D_), jnp.float32)
        cnorm_scratch[...] = jax.lax.dot_general(
            ones, cb * cb, (((1,), (1,)), ((), ())),
            precision=_HI, preferred_element_type=jnp.float32)
        p0, p1, p2 = _split3(cb)
        c0_s[...] = p0
        c1_s[...] = p1
        c2_s[...] = p2

    r = r_scratch[pl.ds(b * BT_, BT_), :]
    r0, r1, r2 = _split3(r)
    c0 = c0_s[...]
    c1 = c1_s[...]
    c2 = c2_s[...]
    rc = (_nt(r0, c0) + _nt(r0, c1) + _nt(r1, c0)
          + _nt(r0, c2) + _nt(r1, c1) + _nt(r2, c0))
    s = cnorm_scratch[...] - 2.0 * rc

    m = jnp.min(s, axis=1, keepdims=True)
    kiota = jax.lax.broadcasted_iota(jnp.int32, (BT_, K_), 1)
    idx = jnp.min(jnp.where(s == m, kiota, K_), axis=1)

    onehot = (kiota == idx[:, None]).astype(jnp.bfloat16)
    chosen = _nn(onehot, c0) + _nn(onehot, c1) + _nn(onehot, c2)
    r_new = r - chosen
    r_scratch[pl.ds(b * BT_, BT_), :] = r_new
    r_out_ref[...] = r_new
    ids_ref[...] = idx.reshape(1, 1, BT_)


def kernel(preference_vector, codebooks):
    ids_lb, residual = pl.pallas_call(
        _rvq_body,
        grid=(NUM_LEVELS_, NBT_),
        in_specs=[
            pl.BlockSpec((BT_, D_), lambda l, b: (b, 0)),
            pl.BlockSpec((1, K_, D_), lambda l, b: (l, 0, 0)),
        ],
        out_specs=[
            pl.BlockSpec((1, 1, BT_), lambda l, b: (l, 0, b)),
            pl.BlockSpec((BT_, D_), lambda l, b: (b, 0)),
        ],
        out_shape=[
            jax.ShapeDtypeStruct((NUM_LEVELS_, 1, B_), jnp.int32),
            jax.ShapeDtypeStruct((B_, D_), jnp.float32),
        ],
        scratch_shapes=[
            pltpu.VMEM((B_, D_), jnp.float32),
            pltpu.VMEM((1, K_), jnp.float32),
            pltpu.VMEM((K_, D_), jnp.bfloat16),
            pltpu.VMEM((K_, D_), jnp.bfloat16),
            pltpu.VMEM((K_, D_), jnp.bfloat16),
        ],
    )(preference_vector, codebooks)
    ids = ids_lb.reshape(NUM_LEVELS_, B_).T
    return ids, residual


# speedup vs baseline: 19.0959x; 1.0610x over previous
import jax
import jax.numpy as jnp
from jax.experimental import pallas as pl
from jax.experimental.pallas import tpu as pltpu

NUM_LEVELS_ = 8
K_ = 8192
D_ = 256
B_ = 4096
BT_ = 256
NBT_ = B_ // BT_

_HI = jax.lax.Precision.HIGHEST


def _nt(a, b):
    return jax.lax.dot_general(a, b, (((1,), (1,)), ((), ())),
                               preferred_element_type=jnp.float32)


def _nn(a, b):
    return jax.lax.dot_general(a, b, (((1,), (0,)), ((), ())),
                               preferred_element_type=jnp.float32)


def _split3(x):
    x0 = x.astype(jnp.bfloat16)
    rem = x - x0.astype(jnp.float32)
    x1 = rem.astype(jnp.bfloat16)
    x2 = (rem - x1.astype(jnp.float32)).astype(jnp.bfloat16)
    return x0, x1, x2


def _rvq_body(r_in_ref, cb_ref, ids_ref, r_out_ref,
              r_scratch, cnorm_scratch, c0_s, c1_s, c2_s):
    l = pl.program_id(0)
    b = pl.program_id(1)

    @pl.when(l == 0)
    def _init_residual():
        r_scratch[pl.ds(b * BT_, BT_), :] = r_in_ref[...]

    @pl.when(b == 0)
    def _per_level_prep():
        cb = cb_ref[0]
        ones = jnp.ones((1, D_), jnp.float32)
        cnorm_scratch[...] = jax.lax.dot_general(
            ones, cb * cb, (((1,), (1,)), ((), ())),
            precision=_HI, preferred_element_type=jnp.float32)
        p0, p1, p2 = _split3(cb)
        c0_s[...] = p0
        c1_s[...] = p1
        c2_s[...] = p2

    r = r_scratch[pl.ds(b * BT_, BT_), :]
    r0, r1, r2 = _split3(r)
    c0 = c0_s[...]
    c1 = c1_s[...]
    c2 = c2_s[...]
    rc = (_nt(r0, c0) + _nt(r0, c1) + _nt(r1, c0)
          + _nt(r0, c2) + _nt(r2, c0))
    s = cnorm_scratch[...] - 2.0 * rc

    m = jnp.min(s, axis=1, keepdims=True)
    kiota = jax.lax.broadcasted_iota(jnp.int32, (BT_, K_), 1)
    idx = jnp.min(jnp.where(s == m, kiota, K_), axis=1)

    onehot = (kiota == idx[:, None]).astype(jnp.bfloat16)
    chosen = _nn(onehot, c0) + _nn(onehot, c1) + _nn(onehot, c2)
    r_new = r - chosen
    r_scratch[pl.ds(b * BT_, BT_), :] = r_new
    r_out_ref[...] = r_new
    ids_ref[...] = idx.reshape(1, 1, BT_)


def kernel(preference_vector, codebooks):
    ids_lb, residual = pl.pallas_call(
        _rvq_body,
        grid=(NUM_LEVELS_, NBT_),
        in_specs=[
            pl.BlockSpec((BT_, D_), lambda l, b: (b, 0)),
            pl.BlockSpec((1, K_, D_), lambda l, b: (l, 0, 0)),
        ],
        out_specs=[
            pl.BlockSpec((1, 1, BT_), lambda l, b: (l, 0, b)),
            pl.BlockSpec((BT_, D_), lambda l, b: (b, 0)),
        ],
        out_shape=[
            jax.ShapeDtypeStruct((NUM_LEVELS_, 1, B_), jnp.int32),
            jax.ShapeDtypeStruct((B_, D_), jnp.float32),
        ],
        scratch_shapes=[
            pltpu.VMEM((B_, D_), jnp.float32),
            pltpu.VMEM((1, K_), jnp.float32),
            pltpu.VMEM((K_, D_), jnp.bfloat16),
            pltpu.VMEM((K_, D_), jnp.bfloat16),
            pltpu.VMEM((K_, D_), jnp.bfloat16),
        ],
    )(preference_vector, codebooks)
    ids = ids_lb.reshape(NUM_LEVELS_, B_).T
    return ids, residual


# speedup vs baseline: 27.2426x; 1.4266x over previous
import jax
import jax.numpy as jnp
from jax.experimental import pallas as pl
from jax.experimental.pallas import tpu as pltpu

NUM_LEVELS_ = 8
K_ = 8192
D_ = 256
B_ = 4096
BT_ = 256
NBT_ = B_ // BT_
NT_ = NUM_LEVELS_ * NBT_

_HI = jax.lax.Precision.HIGHEST


def _nt(a, b):
    return jax.lax.dot_general(a, b, (((1,), (1,)), ((), ())),
                               preferred_element_type=jnp.float32)


def _nn(a, b):
    return jax.lax.dot_general(a, b, (((1,), (0,)), ((), ())),
                               preferred_element_type=jnp.float32)


def _split3(x):
    x0 = x.astype(jnp.bfloat16)
    rem = x - x0.astype(jnp.float32)
    x1 = rem.astype(jnp.bfloat16)
    x2 = (rem - x1.astype(jnp.float32)).astype(jnp.bfloat16)
    return x0, x1, x2


def _rvq_body(r_in_ref, cb_ref, ids_ref, r_out_ref,
              r_scratch, cnorm_scratch, c0_s, c1_s, c2_s, idx_s):
    t = pl.program_id(0)
    b = t % NBT_
    bp = (t + NT_ - 1) % NBT_

    def extract_prev():
        idxp = idx_s[(t + 1) % 2, 0, :]
        kiota = jax.lax.broadcasted_iota(jnp.int32, (BT_, K_), 1)
        onehot = (kiota == idxp[:, None]).astype(jnp.bfloat16)
        chosen = (_nn(onehot, c0_s[...]) + _nn(onehot, c1_s[...])
                  + _nn(onehot, c2_s[...]))
        rp = r_scratch[pl.ds(bp * BT_, BT_), :]
        r_new = rp - chosen
        r_scratch[pl.ds(bp * BT_, BT_), :] = r_new
        r_out_ref[...] = r_new
        ids_ref[...] = idxp.reshape(1, 1, BT_)

    @pl.when(jnp.logical_and(t > 0, b == 0))
    def _extract_at_boundary():
        extract_prev()

    @pl.when(jnp.logical_and(b == 0, t < NT_))
    def _per_level_prep():
        cb = cb_ref[0]
        ones = jnp.ones((1, D_), jnp.float32)
        cnorm_scratch[...] = jax.lax.dot_general(
            ones, cb * cb, (((1,), (1,)), ((), ())),
            precision=_HI, preferred_element_type=jnp.float32)
        p0, p1, p2 = _split3(cb)
        c0_s[...] = p0
        c1_s[...] = p1
        c2_s[...] = p2

    @pl.when(t < NT_)
    def _scores_and_argmin():
        @pl.when(t < NBT_)
        def _init_residual():
            r_scratch[pl.ds(b * BT_, BT_), :] = r_in_ref[...]

        r = r_scratch[pl.ds(b * BT_, BT_), :]
        r0, r1, r2 = _split3(r)
        c0 = c0_s[...]
        c1 = c1_s[...]
        c2 = c2_s[...]
        rc = (_nt(r0, c0) + _nt(r0, c1) + _nt(r1, c0)
              + _nt(r0, c2) + _nt(r1, c1) + _nt(r2, c0))
        s = cnorm_scratch[...] - 2.0 * rc

        m = jnp.min(s, axis=1, keepdims=True)
        kiota = jax.lax.broadcasted_iota(jnp.int32, (BT_, K_), 1)
        idx = jnp.min(jnp.where(s == m, kiota, K_), axis=1)
        idx_s[t % 2, 0, :] = idx

    @pl.when(jnp.logical_and(t > 0, b != 0))
    def _extract_overlapped():
        extract_prev()


def kernel(preference_vector, codebooks):
    ids_lb, residual = pl.pallas_call(
        _rvq_body,
        grid=(NT_ + 1,),
        in_specs=[
            pl.BlockSpec((BT_, D_), lambda t: (t % NBT_, 0)),
            pl.BlockSpec((1, K_, D_), lambda t: ((t % NT_) // NBT_, 0, 0)),
        ],
        out_specs=[
            pl.BlockSpec(
                (1, 1, BT_),
                lambda t: (((t + NT_ - 1) // NBT_) % NUM_LEVELS_, 0,
                           (t + NT_ - 1) % NBT_)),
            pl.BlockSpec((BT_, D_), lambda t: ((t + NT_ - 1) % NBT_, 0)),
        ],
        out_shape=[
            jax.ShapeDtypeStruct((NUM_LEVELS_, 1, B_), jnp.int32),
            jax.ShapeDtypeStruct((B_, D_), jnp.float32),
        ],
        scratch_shapes=[
            pltpu.VMEM((B_, D_), jnp.float32),
            pltpu.VMEM((1, K_), jnp.float32),
            pltpu.VMEM((K_, D_), jnp.bfloat16),
            pltpu.VMEM((K_, D_), jnp.bfloat16),
            pltpu.VMEM((K_, D_), jnp.bfloat16),
            pltpu.VMEM((2, 1, BT_), jnp.int32),
        ],
    )(preference_vector, codebooks)
    ids = ids_lb.reshape(NUM_LEVELS_, B_).T
    return ids, residual


# speedup vs baseline: 28.1591x; 1.0336x over previous
import jax
import jax.numpy as jnp
from jax.experimental import pallas as pl
from jax.experimental.pallas import tpu as pltpu

NUM_LEVELS_ = 8
K_ = 8192
D_ = 256
B_ = 4096
BT_ = 256
NBT_ = B_ // BT_
NT_ = NUM_LEVELS_ * NBT_

_HI = jax.lax.Precision.HIGHEST


def _nt(a, b):
    return jax.lax.dot_general(a, b, (((1,), (1,)), ((), ())),
                               preferred_element_type=jnp.float32)


def _nn(a, b):
    return jax.lax.dot_general(a, b, (((1,), (0,)), ((), ())),
                               preferred_element_type=jnp.float32)


def _split3(x):
    x0 = x.astype(jnp.bfloat16)
    rem = x - x0.astype(jnp.float32)
    x1 = rem.astype(jnp.bfloat16)
    x2 = (rem - x1.astype(jnp.float32)).astype(jnp.bfloat16)
    return x0, x1, x2


def _rvq_body(r_in_ref, cb_ref, ids_ref, r_out_ref,
              r_scratch, cnorm_scratch, c0_s, c1_s, c2_s, idx_s):
    t = pl.program_id(0)
    b = t % NBT_
    bp = (t + NT_ - 1) % NBT_

    def extract_prev():
        idxp = idx_s[(t + 1) % 2, 0, :]
        kiota = jax.lax.broadcasted_iota(jnp.int32, (BT_, K_), 1)
        onehot = (kiota == idxp[:, None]).astype(jnp.bfloat16)
        chosen = (_nn(onehot, c0_s[...]) + _nn(onehot, c1_s[...])
                  + _nn(onehot, c2_s[...]))
        rp = r_scratch[pl.ds(bp * BT_, BT_), :]
        r_new = rp - chosen
        r_scratch[pl.ds(bp * BT_, BT_), :] = r_new
        r_out_ref[...] = r_new
        ids_ref[...] = idxp.reshape(1, 1, BT_)

    @pl.when(jnp.logical_and(t > 0, b == 0))
    def _extract_at_boundary():
        extract_prev()

    @pl.when(jnp.logical_and(b == 0, t < NT_))
    def _per_level_prep():
        cb = cb_ref[0]
        ones = jnp.ones((1, D_), jnp.float32)
        cnorm_scratch[...] = jax.lax.dot_general(
            ones, cb * cb, (((1,), (1,)), ((), ())),
            precision=_HI, preferred_element_type=jnp.float32)
        p0, p1, p2 = _split3(cb)
        c0_s[...] = p0
        c1_s[...] = p1
        c2_s[...] = p2

    @pl.when(t < NT_)
    def _scores_and_argmin():
        @pl.when(t < NBT_)
        def _init_residual():
            r_scratch[pl.ds(b * BT_, BT_), :] = r_in_ref[...]

        r = r_scratch[pl.ds(b * BT_, BT_), :]
        r0, r1, r2 = _split3(r)
        c0 = c0_s[...]
        c1 = c1_s[...]
        c2 = c2_s[...]
        rc = (_nt(r0, c0) + _nt(r0, c1) + _nt(r1, c0)
              + _nt(r0, c2) + _nt(r1, c1) + _nt(r2, c0))
        s = cnorm_scratch[...] - 2.0 * rc

        idx = jnp.argmin(s, axis=1).astype(jnp.int32)
        idx_s[t % 2, 0, :] = idx

    @pl.when(jnp.logical_and(t > 0, b != 0))
    def _extract_overlapped():
        extract_prev()


def kernel(preference_vector, codebooks):
    ids_lb, residual = pl.pallas_call(
        _rvq_body,
        grid=(NT_ + 1,),
        in_specs=[
            pl.BlockSpec((BT_, D_), lambda t: (t % NBT_, 0)),
            pl.BlockSpec((1, K_, D_), lambda t: ((t % NT_) // NBT_, 0, 0)),
        ],
        out_specs=[
            pl.BlockSpec(
                (1, 1, BT_),
                lambda t: (((t + NT_ - 1) // NBT_) % NUM_LEVELS_, 0,
                           (t + NT_ - 1) % NBT_)),
            pl.BlockSpec((BT_, D_), lambda t: ((t + NT_ - 1) % NBT_, 0)),
        ],
        out_shape=[
            jax.ShapeDtypeStruct((NUM_LEVELS_, 1, B_), jnp.int32),
            jax.ShapeDtypeStruct((B_, D_), jnp.float32),
        ],
        scratch_shapes=[
            pltpu.VMEM((B_, D_), jnp.float32),
            pltpu.VMEM((1, K_), jnp.float32),
            pltpu.VMEM((K_, D_), jnp.bfloat16),
            pltpu.VMEM((K_, D_), jnp.bfloat16),
            pltpu.VMEM((K_, D_), jnp.bfloat16),
            pltpu.VMEM((2, 1, BT_), jnp.int32),
        ],
    )(preference_vector, codebooks)
    ids = ids_lb.reshape(NUM_LEVELS_, B_).T
    return ids, residual
